# baseline (speedup 1.0000x reference)
import functools

import jax
import jax.numpy as jnp
from jax import lax
from jax.experimental import pallas as pl
from jax.experimental.pallas import tpu as pltpu
from jax.experimental.pallas import tpu_sc as plsc

_NC = 2
_NS = 16
_NW = _NC * _NS
_L = 16

_B = 16384
_R = 32
_BPW = _B // _NW
_CH = 256


def _sc_body(idx_hbm, eps_hbm, tab, u_hbm,
             idx_v, g_v, eps_v, sem_g, sem_e):
    wid = lax.axis_index("s") * _NC + lax.axis_index("c")
    for h in range(_BPW // _CH):
        row0 = wid * _BPW + h * _CH
        pltpu.sync_copy(idx_hbm.at[pl.ds(row0, _CH)], idx_v)
        cp_e = pltpu.async_copy(eps_hbm.at[pl.ds(row0, _CH)], eps_v, sem_e)

        def fire(g, _):
            vec = idx_v[pl.ds(g * _L, _L)]
            for u in range(_L):
                idx = vec[u]
                j = g * _L + u
                pltpu.async_copy(tab.at[idx], g_v.at[j], sem_g)
            return 0

        lax.fori_loop(0, _CH // _L, fire, 0)
        pltpu.make_async_copy(tab.at[pl.ds(0, _CH)], g_v, sem_g).wait()
        cp_e.wait()

        def compute(r, _):
            for c in range(_R // _L):
                sl = pl.ds(c * _L, _L)
                lam_sl = pl.ds(_R + c * _L, _L)
                eps_v[r, sl] = (g_v[r, sl]
                                + eps_v[r, sl]
                                * jnp.exp(g_v[r, lam_sl] * 0.5))
            return 0

        lax.fori_loop(0, _CH, compute, 0, unroll=4)
        pltpu.sync_copy(eps_v, u_hbm.at[pl.ds(row0, _CH)])


_sc_gather_mode = functools.partial(
    pl.kernel,
    out_type=jax.ShapeDtypeStruct((_B, _R), jnp.float32),
    mesh=plsc.VectorSubcoreMesh(core_axis_name="c", subcore_axis_name="s"),
    scratch_types=[
        pltpu.VMEM((_CH,), jnp.int32),
        pltpu.VMEM((_CH, 2 * _R), jnp.float32),
        pltpu.VMEM((_CH, _R), jnp.float32),
        pltpu.SemaphoreType.DMA,
        pltpu.SemaphoreType.DMA,
    ],
)(_sc_body)


_TB = 8192


def _mlp_body(u0, u1, u2, w_t, b_in, w_m, w_l, b_m, b_l, mean_ref, lv_ref):
    u = jnp.concatenate([u0[...], u1[...], u2[...]], axis=1)
    h = jnp.dot(u, w_t[...], preferred_element_type=jnp.float32) + b_in[...]
    h = jnp.tanh(h)
    mean_ref[...] = lax.dot_general(
        w_m[...], h, (((1,), (1,)), ((), ())),
        preferred_element_type=jnp.float32) + b_m[...]
    lv_ref[...] = lax.dot_general(
        w_l[...], h, (((1,), (1,)), ((), ())),
        preferred_element_type=jnp.float32) + b_l[...]


def _mlp(u0, u1, u2, w_t, b_in, w_m, w_l, b_m, b_l):
    return pl.pallas_call(
        _mlp_body,
        grid=(_B // _TB,),
        in_specs=[
            pl.BlockSpec((_TB, _R), lambda i: (i, 0)),
            pl.BlockSpec((_TB, _R), lambda i: (i, 0)),
            pl.BlockSpec((_TB, _R), lambda i: (i, 0)),
            pl.BlockSpec((3 * _R, 128), lambda i: (0, 0)),
            pl.BlockSpec((1, 128), lambda i: (0, 0)),
            pl.BlockSpec((1, 128), lambda i: (0, 0)),
            pl.BlockSpec((1, 128), lambda i: (0, 0)),
            pl.BlockSpec((1, 1), lambda i: (0, 0)),
            pl.BlockSpec((1, 1), lambda i: (0, 0)),
        ],
        out_specs=[
            pl.BlockSpec((1, _TB), lambda i: (0, i)),
            pl.BlockSpec((1, _TB), lambda i: (0, i)),
        ],
        out_shape=[
            jax.ShapeDtypeStruct((1, _B), jnp.float32),
            jax.ShapeDtypeStruct((1, _B), jnp.float32),
        ],
    )(u0, u1, u2, w_t, b_in, w_m, w_l, b_m, b_l)


def kernel(x, mu0, mu1, mu2, lam0, lam1, lam2, eps0, eps1, eps2,
           W_in, b_in, W_mean, b_mean, W_lv, b_lv):
    xi = x.astype(jnp.int32)
    us = []
    for m, (mu_t, lam_t, eps_m) in enumerate(
            ((mu0, lam0, eps0), (mu1, lam1, eps1), (mu2, lam2, eps2))):
        tab = jnp.concatenate([mu_t, lam_t], axis=1)
        us.append(_sc_gather_mode(xi[:, m], eps_m, tab))
    w_t = W_in.T
    mean, log_var = _mlp(us[0], us[1], us[2], w_t, b_in.reshape(1, 128),
                         W_mean, W_lv,
                         b_mean.reshape(1, 1), b_lv.reshape(1, 1))
    return (mean.reshape(_B, 1), log_var.reshape(_B, 1))


# speedup vs baseline: 1.1052x; 1.1052x over previous
import functools

import jax
import jax.numpy as jnp
from jax import lax
from jax.experimental import pallas as pl
from jax.experimental.pallas import tpu as pltpu
from jax.experimental.pallas import tpu_sc as plsc

_NC = 2
_NS = 16
_NW = _NC * _NS
_L = 16

_B = 16384
_R = 32
_BPW = _B // _NW
_CH = 256


def _sc_body(idx_hbm, eps_hbm, mu_t, lam_t, u_hbm,
             idx_v, mu_g, lam_g, eps_v, sem_g, sem_e):
    wid = lax.axis_index("s") * _NC + lax.axis_index("c")
    for h in range(_BPW // _CH):
        row0 = wid * _BPW + h * _CH
        pltpu.sync_copy(idx_hbm.at[pl.ds(row0, _CH)], idx_v)
        cp_e = pltpu.async_copy(eps_hbm.at[pl.ds(row0, _CH)], eps_v, sem_e)

        def fire(g, _):
            vec = idx_v[pl.ds(g * _L, _L)]
            for u in range(_L):
                idx = vec[u]
                j = g * _L + u
                pltpu.async_copy(mu_t.at[idx], mu_g.at[j], sem_g)
                pltpu.async_copy(lam_t.at[idx], lam_g.at[j], sem_g)
            return 0

        lax.fori_loop(0, _CH // _L, fire, 0)
        pltpu.make_async_copy(mu_t.at[pl.ds(0, _CH)], mu_g, sem_g).wait()
        pltpu.make_async_copy(lam_t.at[pl.ds(0, _CH)], lam_g, sem_g).wait()
        cp_e.wait()

        def compute(r, _):
            for c in range(_R // _L):
                sl = pl.ds(c * _L, _L)
                eps_v[r, sl] = (mu_g[r, sl]
                                + eps_v[r, sl] * jnp.exp(lam_g[r, sl] * 0.5))
            return 0

        lax.fori_loop(0, _CH, compute, 0, unroll=4)
        pltpu.sync_copy(eps_v, u_hbm.at[pl.ds(row0, _CH)])


_sc_gather_mode = functools.partial(
    pl.kernel,
    out_type=jax.ShapeDtypeStruct((_B, _R), jnp.float32),
    mesh=plsc.VectorSubcoreMesh(core_axis_name="c", subcore_axis_name="s"),
    scratch_types=[
        pltpu.VMEM((_CH,), jnp.int32),
        pltpu.VMEM((_CH, _R), jnp.float32),
        pltpu.VMEM((_CH, _R), jnp.float32),
        pltpu.VMEM((_CH, _R), jnp.float32),
        pltpu.SemaphoreType.DMA,
        pltpu.SemaphoreType.DMA,
    ],
)(_sc_body)


_TB = 8192


def _mlp_body(u0, u1, u2, w_t, b_in, w_m, w_l, b_m, b_l, mean_ref, lv_ref):
    u = jnp.concatenate([u0[...], u1[...], u2[...]], axis=1)
    h = jnp.dot(u, w_t[...], preferred_element_type=jnp.float32) + b_in[...]
    h = jnp.tanh(h)
    mean_ref[...] = lax.dot_general(
        w_m[...], h, (((1,), (1,)), ((), ())),
        preferred_element_type=jnp.float32) + b_m[...]
    lv_ref[...] = lax.dot_general(
        w_l[...], h, (((1,), (1,)), ((), ())),
        preferred_element_type=jnp.float32) + b_l[...]


def _mlp(u0, u1, u2, w_t, b_in, w_m, w_l, b_m, b_l):
    return pl.pallas_call(
        _mlp_body,
        grid=(_B // _TB,),
        in_specs=[
            pl.BlockSpec((_TB, _R), lambda i: (i, 0)),
            pl.BlockSpec((_TB, _R), lambda i: (i, 0)),
            pl.BlockSpec((_TB, _R), lambda i: (i, 0)),
            pl.BlockSpec((3 * _R, 128), lambda i: (0, 0)),
            pl.BlockSpec((1, 128), lambda i: (0, 0)),
            pl.BlockSpec((1, 128), lambda i: (0, 0)),
            pl.BlockSpec((1, 128), lambda i: (0, 0)),
            pl.BlockSpec((1, 1), lambda i: (0, 0)),
            pl.BlockSpec((1, 1), lambda i: (0, 0)),
        ],
        out_specs=[
            pl.BlockSpec((1, _TB), lambda i: (0, i)),
            pl.BlockSpec((1, _TB), lambda i: (0, i)),
        ],
        out_shape=[
            jax.ShapeDtypeStruct((1, _B), jnp.float32),
            jax.ShapeDtypeStruct((1, _B), jnp.float32),
        ],
    )(u0, u1, u2, w_t, b_in, w_m, w_l, b_m, b_l)


def kernel(x, mu0, mu1, mu2, lam0, lam1, lam2, eps0, eps1, eps2,
           W_in, b_in, W_mean, b_mean, W_lv, b_lv):
    xi = x.astype(jnp.int32)
    us = []
    for m, (mu_t, lam_t, eps_m) in enumerate(
            ((mu0, lam0, eps0), (mu1, lam1, eps1), (mu2, lam2, eps2))):
        us.append(_sc_gather_mode(xi[:, m], eps_m, mu_t, lam_t))
    w_t = W_in.T
    mean, log_var = _mlp(us[0], us[1], us[2], w_t, b_in.reshape(1, 128),
                         W_mean, W_lv,
                         b_mean.reshape(1, 1), b_lv.reshape(1, 1))
    return (mean.reshape(_B, 1), log_var.reshape(_B, 1))
